# baseline (speedup 1.0000x reference)
import numpy as np

import jax
import jax.numpy as jnp
from jax import lax
from jax.experimental import pallas as pl
from jax.experimental.pallas import tpu as pltpu
from jax.experimental.pallas import tpu_sc as plsc

_N = 500000
_B = 1024
_EPS = 1e-05
_TSCALE = 2.0

_PN = 524288
_W = _PN // 32
_GRP = 512
_NG = _W // _GRP
_RT = 1152
_TCROWS = (3 * _PN) // 1024
_TCBLK = _TCROWS // 6


def _wrap(x):
    return jnp.remainder(x + 0.5, 1.0) - 0.5


def _const_table():
    lin = [np.arange(16, dtype=np.int32) + 16 * j for j in range(3)]
    cbn = np.zeros((16, 16), np.int32)
    for j in range(3):
        cbn[j] = lin[j] // 3
        cbn[3 + j] = lin[j] % 3
    cbn[9] = np.arange(16, dtype=np.int32)
    return cbn


def _sc1_body(cb_hbm, idx_hbm, ev_hbm, er_hbm, out_hbm,
              cb, idxw, evb, erb, acc):
    cid = lax.axis_index("c")
    sid = lax.axis_index("s")
    wid = cid * 16 + sid
    zeros16 = jnp.zeros((16,), jnp.float32)
    ones16 = jnp.ones((16,), jnp.float32)

    def zero_body(r, carry):
        acc[pl.ds(r * 16, 16)] = zeros16
        return carry

    lax.fori_loop(0, _RT, zero_body, 0)
    pltpu.sync_copy(cb_hbm, cb)
    pltpu.sync_copy(idx_hbm.at[pl.ds(wid * _W, _W)], idxw)

    def group_body(g, carry):
        base = wid * _W + g * _GRP
        pltpu.sync_copy(ev_hbm.at[pl.ds(base * 3, _GRP * 3)], evb)
        pltpu.sync_copy(er_hbm.at[pl.ds(base * 3, _GRP * 3)], erb)
        for s in range(_GRP // 16):
            lb = g * _GRP + s * 16
            for j in range(3):
                idx3 = plsc.load_gather(idxw, [cb[j] + lb])
                tgt = idx3 * 16 + cb[3 + j]
                evj = evb[pl.ds(s * 48 + 16 * j, 16)]
                erj = erb[pl.ds(s * 48 + 16 * j, 16)]
                plsc.addupdate_scatter(acc, [tgt], evj)
                plsc.addupdate_scatter(acc, [tgt + 3], erj)
            idx16 = idxw[pl.ds(lb, 16)]
            plsc.addupdate_scatter(acc, [idx16 * 16 + 6], ones16)
        return carry

    lax.fori_loop(0, _NG, group_body, 0)
    pltpu.sync_copy(acc, out_hbm.at[wid])


_sc_mesh = plsc.VectorSubcoreMesh(core_axis_name="c", subcore_axis_name="s")

_sc1 = pl.kernel(
    _sc1_body,
    out_type=jax.ShapeDtypeStruct((32, _RT * 16), jnp.float32),
    mesh=_sc_mesh,
    compiler_params=pltpu.CompilerParams(needs_layout_passes=False),
    scratch_types=[
        pltpu.VMEM((16, 16), jnp.int32),
        pltpu.VMEM((_W,), jnp.int32),
        pltpu.VMEM((_GRP * 3,), jnp.float32),
        pltpu.VMEM((_GRP * 3,), jnp.float32),
        pltpu.VMEM((_RT * 16,), jnp.float32),
    ],
    name="sc_segment_sums",
)


def _mred_body(p_ref, m_ref):
    s = jnp.sum(p_ref[...], axis=0)
    cnt = jnp.maximum(s[:, 6:7], 1.0)
    m_ref[...] = s / cnt


_mred = pl.pallas_call(
    _mred_body,
    out_shape=jax.ShapeDtypeStruct((_RT, 16), jnp.float32),
)


def _sc2_body(cb_hbm, idx_hbm, t_hbm, ev_hbm, er_hbm, m_hbm,
              evc_hbm, erc_hbm, t3_hbm,
              cb, idxw, tw, means, evb, erb, evcb, ercb, t3b):
    cid = lax.axis_index("c")
    sid = lax.axis_index("s")
    wid = cid * 16 + sid
    pltpu.sync_copy(cb_hbm, cb)
    pltpu.sync_copy(m_hbm, means)
    pltpu.sync_copy(idx_hbm.at[pl.ds(wid * _W, _W)], idxw)
    pltpu.sync_copy(t_hbm.at[pl.ds(wid * _W, _W)], tw)

    def group_body(g, carry):
        base = wid * _W + g * _GRP
        pltpu.sync_copy(ev_hbm.at[pl.ds(base * 3, _GRP * 3)], evb)
        pltpu.sync_copy(er_hbm.at[pl.ds(base * 3, _GRP * 3)], erb)
        for s in range(_GRP // 16):
            lb = g * _GRP + s * 16
            for j in range(3):
                rows = cb[j] + lb
                idx3 = plsc.load_gather(idxw, [rows])
                t3v = plsc.load_gather(tw, [rows])
                mi = idx3 * 16 + cb[3 + j]
                mv = plsc.load_gather(means, [mi])
                mr = plsc.load_gather(means, [mi + 3])
                evj = evb[pl.ds(s * 48 + 16 * j, 16)]
                erj = erb[pl.ds(s * 48 + 16 * j, 16)]
                evcb[pl.ds(s * 48 + 16 * j, 16)] = evj - mv
                ercb[pl.ds(s * 48 + 16 * j, 16)] = erj - mr
                t3b[pl.ds(s * 48 + 16 * j, 16)] = t3v
        pltpu.sync_copy(evcb, evc_hbm.at[pl.ds(base * 3, _GRP * 3)])
        pltpu.sync_copy(ercb, erc_hbm.at[pl.ds(base * 3, _GRP * 3)])
        pltpu.sync_copy(t3b, t3_hbm.at[pl.ds(base * 3, _GRP * 3)])
        return carry

    lax.fori_loop(0, _NG, group_body, 0)


_sc2 = pl.kernel(
    _sc2_body,
    out_type=(
        jax.ShapeDtypeStruct((3 * _PN,), jnp.float32),
        jax.ShapeDtypeStruct((3 * _PN,), jnp.float32),
        jax.ShapeDtypeStruct((3 * _PN,), jnp.float32),
    ),
    mesh=_sc_mesh,
    compiler_params=pltpu.CompilerParams(needs_layout_passes=False),
    scratch_types=[
        pltpu.VMEM((16, 16), jnp.int32),
        pltpu.VMEM((_W,), jnp.int32),
        pltpu.VMEM((_W,), jnp.float32),
        pltpu.VMEM((_RT * 16,), jnp.float32),
        pltpu.VMEM((_GRP * 3,), jnp.float32),
        pltpu.VMEM((_GRP * 3,), jnp.float32),
        pltpu.VMEM((_GRP * 3,), jnp.float32),
        pltpu.VMEM((_GRP * 3,), jnp.float32),
        pltpu.VMEM((_GRP * 3,), jnp.float32),
    ],
    name="sc_center_expand",
)


def _tc_body(t3_ref, f0_ref, v0_ref, evc_ref, erc_ref, ft_ref, vt_ref, rt_ref):
    t = _TSCALE * t3_ref[...]
    f0 = f0_ref[...]
    v0 = v0_ref[...]
    ev = evc_ref[...]
    er = erc_ref[...]
    e = jnp.exp(-t)
    sigma_v = jnp.sqrt(jnp.maximum(1.0 - jnp.exp(-2.0 * t), _EPS))
    vt = e * v0 + sigma_v * ev
    coeff = (1.0 - e) / (1.0 + e)
    mu = coeff * (vt + v0)
    sigma_r = jnp.sqrt(jnp.maximum(2.0 * t + 8.0 / (1.0 + jnp.exp(t)) - 4.0,
                                   _EPS))
    rt = _wrap(mu + sigma_r * er)
    ft = _wrap(_wrap(f0) + rt)
    ft_ref[...] = ft
    vt_ref[...] = vt
    rt_ref[...] = rt


_tc = pl.pallas_call(
    _tc_body,
    out_shape=tuple(jax.ShapeDtypeStruct((_TCROWS, 1024), jnp.float32)
                    for _ in range(3)),
    grid=(6,),
    in_specs=[pl.BlockSpec((_TCBLK, 1024), lambda i: (i, 0)) for _ in range(5)],
    out_specs=[pl.BlockSpec((_TCBLK, 1024), lambda i: (i, 0))
               for _ in range(3)],
)


def kernel(t, f0, index, v0, epsilon_v, epsilon_r):
    pad = _PN - _N
    idxp = jnp.concatenate(
        [index.astype(jnp.int32), jnp.full((pad,), _B, jnp.int32)])
    tp = jnp.concatenate([t, jnp.zeros((pad,), jnp.float32)])
    zpad3 = jnp.zeros((3 * pad,), jnp.float32)
    evf = jnp.concatenate([epsilon_v.reshape(-1), zpad3])
    erf = jnp.concatenate([epsilon_r.reshape(-1), zpad3])
    f0f = jnp.concatenate([f0.reshape(-1), zpad3])
    v0f = jnp.concatenate([v0.reshape(-1), zpad3])
    cb = jnp.asarray(_const_table())

    partials = _sc1(cb, idxp, evf, erf)
    means = _mred(partials.reshape(32, _RT, 16))
    evcf, ercf, t3f = _sc2(cb, idxp, tp, evf, erf, means.reshape(-1))

    ftf, vtf, rtf = _tc(
        t3f.reshape(_TCROWS, 1024),
        f0f.reshape(_TCROWS, 1024),
        v0f.reshape(_TCROWS, 1024),
        evcf.reshape(_TCROWS, 1024),
        ercf.reshape(_TCROWS, 1024),
    )

    def unflat(x):
        return x.reshape(-1)[: 3 * _N].reshape(_N, 3)

    return (unflat(ftf), unflat(vtf), unflat(evcf), unflat(ercf), unflat(rtf))


# speedup vs baseline: 1.0766x; 1.0766x over previous
import numpy as np

import jax
import jax.numpy as jnp
from jax import lax
from jax.experimental import pallas as pl
from jax.experimental.pallas import tpu as pltpu
from jax.experimental.pallas import tpu_sc as plsc

_N = 500000
_B = 1024
_EPS = 1e-05
_TSCALE = 2.0

_W0 = 15632
_S0 = _W0 // 16
_S1 = (_N - 31 * _W0) // 16
_NGF = 30
_RT = 1152
_TCBLK = 262144
_TCGRID = (3 * _N + _TCBLK - 1) // _TCBLK


def _wrap(x):
    return jnp.remainder(x + 0.5, 1.0) - 0.5


def _const_table():
    lin = [np.arange(16, dtype=np.int32) + 16 * j for j in range(3)]
    cbn = np.zeros((16, 16), np.int32)
    for j in range(3):
        cbn[j] = lin[j] // 3
        cbn[3 + j] = lin[j] % 3
    cbn[9] = np.arange(16, dtype=np.int32)
    return cbn


def _tile_range(cid, sid):
    wid = cid * 16 + sid
    base_sub = wid * _S0
    nsub = jnp.where(wid == 31, _S1, _S0)
    return wid, base_sub, nsub


def _sc1_body(cb_hbm, idx_hbm, ev_hbm, er_hbm, out_hbm,
              cb, idxw, evb, erb, acc):
    cid = lax.axis_index("c")
    sid = lax.axis_index("s")
    wid, base_sub, nsub = _tile_range(cid, sid)
    zeros16 = jnp.zeros((16,), jnp.float32)
    ones16 = jnp.ones((16,), jnp.float32)

    def zero_body(r, carry):
        acc[pl.ds(r * 16, 16)] = zeros16
        return carry

    lax.fori_loop(0, _RT, zero_body, 0)
    pltpu.sync_copy(cb_hbm, cb)

    def sum_sub(s, lb):
        for j in range(3):
            idx3 = plsc.load_gather(idxw, [cb[j] + lb])
            tgt = idx3 * 16 + cb[3 + j]
            evj = evb[pl.ds(lb * 3 + 16 * j, 16)]
            erj = erb[pl.ds(lb * 3 + 16 * j, 16)]
            plsc.addupdate_scatter(acc, [tgt], evj)
            plsc.addupdate_scatter(acc, [tgt + 3], erj)
        idx16 = idxw[pl.ds(lb, 16)]
        plsc.addupdate_scatter(acc, [idx16 * 16 + 6], ones16)

    def group_body(g, carry):
        base = (base_sub + g * 32) * 16
        pltpu.sync_copy(idx_hbm.at[pl.ds(base, 512)], idxw)
        pltpu.sync_copy(ev_hbm.at[pl.ds(base * 3, 1536)], evb)
        pltpu.sync_copy(er_hbm.at[pl.ds(base * 3, 1536)], erb)
        for s in range(32):
            sum_sub(s, s * 16)
        return carry

    lax.fori_loop(0, _NGF, group_body, 0)

    def tail_body(s2, carry):
        base = (base_sub + s2) * 16
        pltpu.sync_copy(idx_hbm.at[pl.ds(base, 16)], idxw.at[pl.ds(0, 16)])
        pltpu.sync_copy(ev_hbm.at[pl.ds(base * 3, 48)], evb.at[pl.ds(0, 48)])
        pltpu.sync_copy(er_hbm.at[pl.ds(base * 3, 48)], erb.at[pl.ds(0, 48)])
        sum_sub(0, 0)
        return carry

    lax.fori_loop(_NGF * 32, nsub, tail_body, 0)
    pltpu.sync_copy(acc, out_hbm.at[wid])


_sc_mesh = plsc.VectorSubcoreMesh(core_axis_name="c", subcore_axis_name="s")

_sc1 = pl.kernel(
    _sc1_body,
    out_type=jax.ShapeDtypeStruct((32, _RT * 16), jnp.float32),
    mesh=_sc_mesh,
    compiler_params=pltpu.CompilerParams(needs_layout_passes=False),
    scratch_types=[
        pltpu.VMEM((16, 16), jnp.int32),
        pltpu.VMEM((512,), jnp.int32),
        pltpu.VMEM((1536,), jnp.float32),
        pltpu.VMEM((1536,), jnp.float32),
        pltpu.VMEM((_RT * 16,), jnp.float32),
    ],
    name="sc_segment_sums",
)


def _mred_body(p_ref, m_ref):
    s = jnp.sum(p_ref[...], axis=0)
    cnt = jnp.maximum(s[:, 6:7], 1.0)
    m_ref[...] = s / cnt


_mred = pl.pallas_call(
    _mred_body,
    out_shape=jax.ShapeDtypeStruct((_RT, 16), jnp.float32),
)


def _sc2_body(cb_hbm, idx_hbm, t_hbm, ev_hbm, er_hbm, m_hbm,
              evc_hbm, erc_hbm, t3_hbm,
              cb, idxw, tw, means, evb, erb, evcb, ercb, t3b):
    cid = lax.axis_index("c")
    sid = lax.axis_index("s")
    wid, base_sub, nsub = _tile_range(cid, sid)
    pltpu.sync_copy(cb_hbm, cb)
    pltpu.sync_copy(m_hbm, means)

    def center_sub(s, lb, gbase):
        for j in range(3):
            rows = cb[j] + lb
            idx3 = plsc.load_gather(idxw, [rows])
            t3v = plsc.load_gather(tw, [rows])
            mi = idx3 * 16 + cb[3 + j]
            mv = plsc.load_gather(means, [mi])
            mr = plsc.load_gather(means, [mi + 3])
            evj = evb[pl.ds(lb * 3 + 16 * j, 16)]
            erj = erb[pl.ds(lb * 3 + 16 * j, 16)]
            evcb[pl.ds(lb * 3 + 16 * j, 16)] = evj - mv
            ercb[pl.ds(lb * 3 + 16 * j, 16)] = erj - mr
            t3b[pl.ds(lb * 3 + 16 * j, 16)] = t3v

    def group_body(g, carry):
        base = (base_sub + g * 32) * 16
        pltpu.sync_copy(idx_hbm.at[pl.ds(base, 512)], idxw)
        pltpu.sync_copy(t_hbm.at[pl.ds(base, 512)], tw)
        pltpu.sync_copy(ev_hbm.at[pl.ds(base * 3, 1536)], evb)
        pltpu.sync_copy(er_hbm.at[pl.ds(base * 3, 1536)], erb)
        for s in range(32):
            center_sub(s, s * 16, base)
        pltpu.sync_copy(evcb, evc_hbm.at[pl.ds(base * 3, 1536)])
        pltpu.sync_copy(ercb, erc_hbm.at[pl.ds(base * 3, 1536)])
        pltpu.sync_copy(t3b, t3_hbm.at[pl.ds(base * 3, 1536)])
        return carry

    lax.fori_loop(0, _NGF, group_body, 0)

    def tail_body(s2, carry):
        base = (base_sub + s2) * 16
        pltpu.sync_copy(idx_hbm.at[pl.ds(base, 16)], idxw.at[pl.ds(0, 16)])
        pltpu.sync_copy(t_hbm.at[pl.ds(base, 16)], tw.at[pl.ds(0, 16)])
        pltpu.sync_copy(ev_hbm.at[pl.ds(base * 3, 48)], evb.at[pl.ds(0, 48)])
        pltpu.sync_copy(er_hbm.at[pl.ds(base * 3, 48)], erb.at[pl.ds(0, 48)])
        center_sub(0, 0, base)
        pltpu.sync_copy(evcb.at[pl.ds(0, 48)], evc_hbm.at[pl.ds(base * 3, 48)])
        pltpu.sync_copy(ercb.at[pl.ds(0, 48)], erc_hbm.at[pl.ds(base * 3, 48)])
        pltpu.sync_copy(t3b.at[pl.ds(0, 48)], t3_hbm.at[pl.ds(base * 3, 48)])
        return carry

    lax.fori_loop(_NGF * 32, nsub, tail_body, 0)


_sc2 = pl.kernel(
    _sc2_body,
    out_type=(
        jax.ShapeDtypeStruct((3 * _N,), jnp.float32),
        jax.ShapeDtypeStruct((3 * _N,), jnp.float32),
        jax.ShapeDtypeStruct((3 * _N,), jnp.float32),
    ),
    mesh=_sc_mesh,
    compiler_params=pltpu.CompilerParams(needs_layout_passes=False),
    scratch_types=[
        pltpu.VMEM((16, 16), jnp.int32),
        pltpu.VMEM((512,), jnp.int32),
        pltpu.VMEM((512,), jnp.float32),
        pltpu.VMEM((_RT * 16,), jnp.float32),
        pltpu.VMEM((1536,), jnp.float32),
        pltpu.VMEM((1536,), jnp.float32),
        pltpu.VMEM((1536,), jnp.float32),
        pltpu.VMEM((1536,), jnp.float32),
        pltpu.VMEM((1536,), jnp.float32),
    ],
    name="sc_center_expand",
)


def _tc_body(t3_ref, f0_ref, v0_ref, evc_ref, erc_ref, ft_ref, vt_ref, rt_ref):
    t = _TSCALE * t3_ref[...]
    f0 = f0_ref[...]
    v0 = v0_ref[...]
    ev = evc_ref[...]
    er = erc_ref[...]
    e = jnp.exp(-t)
    sigma_v = jnp.sqrt(jnp.maximum(1.0 - jnp.exp(-2.0 * t), _EPS))
    vt = e * v0 + sigma_v * ev
    coeff = (1.0 - e) / (1.0 + e)
    mu = coeff * (vt + v0)
    sigma_r = jnp.sqrt(jnp.maximum(2.0 * t + 8.0 / (1.0 + jnp.exp(t)) - 4.0,
                                   _EPS))
    rt = _wrap(mu + sigma_r * er)
    ft = _wrap(_wrap(f0) + rt)
    ft_ref[...] = ft
    vt_ref[...] = vt
    rt_ref[...] = rt


_tc = pl.pallas_call(
    _tc_body,
    out_shape=tuple(jax.ShapeDtypeStruct((3 * _N,), jnp.float32)
                    for _ in range(3)),
    grid=(_TCGRID,),
    in_specs=[pl.BlockSpec((_TCBLK,), lambda i: (i,)) for _ in range(5)],
    out_specs=[pl.BlockSpec((_TCBLK,), lambda i: (i,)) for _ in range(3)],
)


def kernel(t, f0, index, v0, epsilon_v, epsilon_r):
    idx = index.astype(jnp.int32)
    evf = epsilon_v.reshape(-1)
    erf = epsilon_r.reshape(-1)
    f0f = f0.reshape(-1)
    v0f = v0.reshape(-1)
    cb = jnp.asarray(_const_table())

    partials = _sc1(cb, idx, evf, erf)
    means = _mred(partials.reshape(32, _RT, 16))
    evcf, ercf, t3f = _sc2(cb, idx, t, evf, erf, means.reshape(-1))
    ftf, vtf, rtf = _tc(t3f, f0f, v0f, evcf, ercf)

    def unflat(x):
        return x.reshape(_N, 3)

    return (unflat(ftf), unflat(vtf), unflat(evcf), unflat(ercf), unflat(rtf))


# speedup vs baseline: 12.5818x; 11.6871x over previous
import numpy as np

import jax
import jax.numpy as jnp
from jax import lax
from jax.experimental import pallas as pl
from jax.experimental.pallas import tpu as pltpu
from jax.experimental.pallas import tpu_sc as plsc

_N = 500000
_B = 1024
_EPS = 1e-05
_TSCALE = 2.0

_W0 = 15632
_S0 = _W0 // 16
_S1 = (_N - 31 * _W0) // 16
_NGF = 30
_RT = 1152
_TCBLK = 262144
_TCGRID = (3 * _N + _TCBLK - 1) // _TCBLK


def _wrap(x):
    return jnp.remainder(x + 0.5, 1.0) - 0.5


def _tile_range(cid, sid):
    wid = cid * 16 + sid
    base_sub = wid * _S0
    nsub = jnp.where(wid == 31, _S1, _S0)
    return wid, base_sub, nsub


def _sc1_body(idx_hbm, ev_hbm, er_hbm, out_hbm,
              idxw, evb, erb, acc):
    cid = lax.axis_index("c")
    sid = lax.axis_index("s")
    wid, base_sub, nsub = _tile_range(cid, sid)
    zeros16 = jnp.zeros((16,), jnp.float32)
    ones16 = jnp.ones((16,), jnp.float32)

    def zero_body(r, carry):
        acc[pl.ds(r * 16, 16)] = zeros16
        return carry

    lax.fori_loop(0, _RT, zero_body, 0)

    def sum_sub(lb):
        idx16 = idxw[pl.ds(lb, 16)]
        ti = idx16 * 16
        for c in range(3):
            evc = evb[pl.ds(c * 512 + lb, 16)]
            erc = erb[pl.ds(c * 512 + lb, 16)]
            plsc.addupdate_scatter(acc, [ti + c], evc)
            plsc.addupdate_scatter(acc, [ti + (3 + c)], erc)
        plsc.addupdate_scatter(acc, [ti + 6], ones16)

    def group_body(g, carry):
        base = (base_sub + g * 32) * 16
        pltpu.sync_copy(idx_hbm.at[pl.ds(base, 512)], idxw)
        for c in range(3):
            pltpu.sync_copy(ev_hbm.at[pl.ds(c * _N + base, 512)],
                            evb.at[pl.ds(c * 512, 512)])
            pltpu.sync_copy(er_hbm.at[pl.ds(c * _N + base, 512)],
                            erb.at[pl.ds(c * 512, 512)])
        for s in range(32):
            sum_sub(s * 16)
        return carry

    lax.fori_loop(0, _NGF, group_body, 0)

    def tail_body(s2, carry):
        base = (base_sub + s2) * 16
        pltpu.sync_copy(idx_hbm.at[pl.ds(base, 16)], idxw.at[pl.ds(0, 16)])
        for c in range(3):
            pltpu.sync_copy(ev_hbm.at[pl.ds(c * _N + base, 16)],
                            evb.at[pl.ds(c * 512, 16)])
            pltpu.sync_copy(er_hbm.at[pl.ds(c * _N + base, 16)],
                            erb.at[pl.ds(c * 512, 16)])
        sum_sub(0)
        return carry

    lax.fori_loop(_NGF * 32, nsub, tail_body, 0)
    pltpu.sync_copy(acc, out_hbm.at[wid])


_sc_mesh = plsc.VectorSubcoreMesh(core_axis_name="c", subcore_axis_name="s")

_sc1 = pl.kernel(
    _sc1_body,
    out_type=jax.ShapeDtypeStruct((32, _RT * 16), jnp.float32),
    mesh=_sc_mesh,
    compiler_params=pltpu.CompilerParams(needs_layout_passes=False),
    scratch_types=[
        pltpu.VMEM((512,), jnp.int32),
        pltpu.VMEM((1536,), jnp.float32),
        pltpu.VMEM((1536,), jnp.float32),
        pltpu.VMEM((_RT * 16,), jnp.float32),
    ],
    name="sc_segment_sums",
)


def _mred_body(p_ref, m_ref):
    s = jnp.sum(p_ref[...], axis=0)
    cnt = jnp.maximum(s[:, 6:7], 1.0)
    m_ref[...] = s / cnt


_mred = pl.pallas_call(
    _mred_body,
    out_shape=jax.ShapeDtypeStruct((_RT, 16), jnp.float32),
)


def _sc2_body(idx_hbm, t_hbm, ev_hbm, er_hbm, m_hbm,
              evc_hbm, erc_hbm, t3_hbm,
              idxw, tw, means, evb, erb, evcb, ercb, t3b):
    cid = lax.axis_index("c")
    sid = lax.axis_index("s")
    wid, base_sub, nsub = _tile_range(cid, sid)
    pltpu.sync_copy(m_hbm, means)

    def center_sub(lb):
        idx16 = idxw[pl.ds(lb, 16)]
        t16 = tw[pl.ds(lb, 16)]
        ti = idx16 * 16
        for c in range(3):
            mv = plsc.load_gather(means, [ti + c])
            mr = plsc.load_gather(means, [ti + (3 + c)])
            evc = evb[pl.ds(c * 512 + lb, 16)]
            erc = erb[pl.ds(c * 512 + lb, 16)]
            evcb[pl.ds(c * 512 + lb, 16)] = evc - mv
            ercb[pl.ds(c * 512 + lb, 16)] = erc - mr
            t3b[pl.ds(c * 512 + lb, 16)] = t16

    def group_body(g, carry):
        base = (base_sub + g * 32) * 16
        pltpu.sync_copy(idx_hbm.at[pl.ds(base, 512)], idxw)
        pltpu.sync_copy(t_hbm.at[pl.ds(base, 512)], tw)
        for c in range(3):
            pltpu.sync_copy(ev_hbm.at[pl.ds(c * _N + base, 512)],
                            evb.at[pl.ds(c * 512, 512)])
            pltpu.sync_copy(er_hbm.at[pl.ds(c * _N + base, 512)],
                            erb.at[pl.ds(c * 512, 512)])
        for s in range(32):
            center_sub(s * 16)
        for c in range(3):
            pltpu.sync_copy(evcb.at[pl.ds(c * 512, 512)],
                            evc_hbm.at[pl.ds(c * _N + base, 512)])
            pltpu.sync_copy(ercb.at[pl.ds(c * 512, 512)],
                            erc_hbm.at[pl.ds(c * _N + base, 512)])
            pltpu.sync_copy(t3b.at[pl.ds(c * 512, 512)],
                            t3_hbm.at[pl.ds(c * _N + base, 512)])
        return carry

    lax.fori_loop(0, _NGF, group_body, 0)

    def tail_body(s2, carry):
        base = (base_sub + s2) * 16
        pltpu.sync_copy(idx_hbm.at[pl.ds(base, 16)], idxw.at[pl.ds(0, 16)])
        pltpu.sync_copy(t_hbm.at[pl.ds(base, 16)], tw.at[pl.ds(0, 16)])
        for c in range(3):
            pltpu.sync_copy(ev_hbm.at[pl.ds(c * _N + base, 16)],
                            evb.at[pl.ds(c * 512, 16)])
            pltpu.sync_copy(er_hbm.at[pl.ds(c * _N + base, 16)],
                            erb.at[pl.ds(c * 512, 16)])
        center_sub(0)
        for c in range(3):
            pltpu.sync_copy(evcb.at[pl.ds(c * 512, 16)],
                            evc_hbm.at[pl.ds(c * _N + base, 16)])
            pltpu.sync_copy(ercb.at[pl.ds(c * 512, 16)],
                            erc_hbm.at[pl.ds(c * _N + base, 16)])
            pltpu.sync_copy(t3b.at[pl.ds(c * 512, 16)],
                            t3_hbm.at[pl.ds(c * _N + base, 16)])
        return carry

    lax.fori_loop(_NGF * 32, nsub, tail_body, 0)


_sc2 = pl.kernel(
    _sc2_body,
    out_type=(
        jax.ShapeDtypeStruct((3 * _N,), jnp.float32),
        jax.ShapeDtypeStruct((3 * _N,), jnp.float32),
        jax.ShapeDtypeStruct((3 * _N,), jnp.float32),
    ),
    mesh=_sc_mesh,
    compiler_params=pltpu.CompilerParams(needs_layout_passes=False),
    scratch_types=[
        pltpu.VMEM((512,), jnp.int32),
        pltpu.VMEM((512,), jnp.float32),
        pltpu.VMEM((_RT * 16,), jnp.float32),
        pltpu.VMEM((1536,), jnp.float32),
        pltpu.VMEM((1536,), jnp.float32),
        pltpu.VMEM((1536,), jnp.float32),
        pltpu.VMEM((1536,), jnp.float32),
        pltpu.VMEM((1536,), jnp.float32),
    ],
    name="sc_center_expand",
)


def _tc_body(t3_ref, f0_ref, v0_ref, evc_ref, erc_ref, ft_ref, vt_ref, rt_ref):
    t = _TSCALE * t3_ref[...]
    f0 = f0_ref[...]
    v0 = v0_ref[...]
    ev = evc_ref[...]
    er = erc_ref[...]
    e = jnp.exp(-t)
    sigma_v = jnp.sqrt(jnp.maximum(1.0 - jnp.exp(-2.0 * t), _EPS))
    vt = e * v0 + sigma_v * ev
    coeff = (1.0 - e) / (1.0 + e)
    mu = coeff * (vt + v0)
    sigma_r = jnp.sqrt(jnp.maximum(2.0 * t + 8.0 / (1.0 + jnp.exp(t)) - 4.0,
                                   _EPS))
    rt = _wrap(mu + sigma_r * er)
    ft = _wrap(_wrap(f0) + rt)
    ft_ref[...] = ft
    vt_ref[...] = vt
    rt_ref[...] = rt


_tc = pl.pallas_call(
    _tc_body,
    out_shape=tuple(jax.ShapeDtypeStruct((3 * _N,), jnp.float32)
                    for _ in range(3)),
    grid=(_TCGRID,),
    in_specs=[pl.BlockSpec((_TCBLK,), lambda i: (i,)) for _ in range(5)],
    out_specs=[pl.BlockSpec((_TCBLK,), lambda i: (i,)) for _ in range(3)],
)


def kernel(t, f0, index, v0, epsilon_v, epsilon_r):
    idx = index.astype(jnp.int32)
    evf = epsilon_v.T.reshape(-1)
    erf = epsilon_r.T.reshape(-1)
    f0f = f0.T.reshape(-1)
    v0f = v0.T.reshape(-1)

    partials = _sc1(idx, evf, erf)
    means = _mred(partials.reshape(32, _RT, 16))
    evcf, ercf, t3f = _sc2(idx, t, evf, erf, means.reshape(-1))
    ftf, vtf, rtf = _tc(t3f, f0f, v0f, evcf, ercf)

    def unflat(x):
        return x.reshape(3, _N).T

    return (unflat(ftf), unflat(vtf), unflat(evcf), unflat(ercf), unflat(rtf))


# speedup vs baseline: 15.1023x; 1.2003x over previous
import numpy as np

import jax
import jax.numpy as jnp
from jax import lax
from jax.experimental import pallas as pl
from jax.experimental.pallas import tpu as pltpu
from jax.experimental.pallas import tpu_sc as plsc

_N = 500000
_B = 1024
_EPS = 1e-05
_TSCALE = 2.0

_W0 = 15632
_S0 = _W0 // 16
_S1 = (_N - 31 * _W0) // 16
_NGF = 15
_RT = 1152
_TCBLK = 262144
_TCGRID = (3 * _N + _TCBLK - 1) // _TCBLK


def _wrap(x):
    return jnp.remainder(x + 0.5, 1.0) - 0.5


def _tile_range(cid, sid):
    wid = cid * 16 + sid
    base_sub = wid * _S0
    nsub = jnp.where(wid == 31, _S1, _S0)
    return wid, base_sub, nsub


def _sc1_body(idx_hbm, ev_hbm, er_hbm, out_hbm,
              idxw, evb, erb, acc):
    cid = lax.axis_index("c")
    sid = lax.axis_index("s")
    wid, base_sub, nsub = _tile_range(cid, sid)
    zeros16 = jnp.zeros((16,), jnp.float32)
    ones16 = jnp.ones((16,), jnp.float32)

    def zero_body(r, carry):
        acc[pl.ds(r * 16, 16)] = zeros16
        return carry

    lax.fori_loop(0, _RT, zero_body, 0)

    def sum_sub(lb):
        idx16 = idxw[pl.ds(lb, 16)]
        ti = idx16 * 16
        for c in range(3):
            evc = evb[pl.ds(c * 1024 + lb, 16)]
            erc = erb[pl.ds(c * 1024 + lb, 16)]
            plsc.addupdate_scatter(acc, [ti + c], evc)
            plsc.addupdate_scatter(acc, [ti + (3 + c)], erc)
        plsc.addupdate_scatter(acc, [ti + 6], ones16)

    def group_body(g, carry):
        base = (base_sub + g * 64) * 16
        pltpu.sync_copy(idx_hbm.at[pl.ds(base, 1024)], idxw)
        for c in range(3):
            pltpu.sync_copy(ev_hbm.at[pl.ds(c * _N + base, 1024)],
                            evb.at[pl.ds(c * 1024, 1024)])
            pltpu.sync_copy(er_hbm.at[pl.ds(c * _N + base, 1024)],
                            erb.at[pl.ds(c * 1024, 1024)])
        for s in range(64):
            sum_sub(s * 16)
        return carry

    lax.fori_loop(0, _NGF, group_body, 0)

    def tail_body(s2, carry):
        base = (base_sub + s2) * 16
        pltpu.sync_copy(idx_hbm.at[pl.ds(base, 16)], idxw.at[pl.ds(0, 16)])
        for c in range(3):
            pltpu.sync_copy(ev_hbm.at[pl.ds(c * _N + base, 16)],
                            evb.at[pl.ds(c * 1024, 16)])
            pltpu.sync_copy(er_hbm.at[pl.ds(c * _N + base, 16)],
                            erb.at[pl.ds(c * 1024, 16)])
        sum_sub(0)
        return carry

    lax.fori_loop(_NGF * 64, nsub, tail_body, 0)
    pltpu.sync_copy(acc, out_hbm.at[wid])


_sc_mesh = plsc.VectorSubcoreMesh(core_axis_name="c", subcore_axis_name="s")

_sc1 = pl.kernel(
    _sc1_body,
    out_type=jax.ShapeDtypeStruct((32, _RT * 16), jnp.float32),
    mesh=_sc_mesh,
    compiler_params=pltpu.CompilerParams(needs_layout_passes=False),
    scratch_types=[
        pltpu.VMEM((1024,), jnp.int32),
        pltpu.VMEM((3072,), jnp.float32),
        pltpu.VMEM((3072,), jnp.float32),
        pltpu.VMEM((_RT * 16,), jnp.float32),
    ],
    name="sc_segment_sums",
)


def _mred_body(p_ref, m_ref):
    s = jnp.sum(p_ref[...], axis=0)
    cnt = jnp.maximum(s[:, 6:7], 1.0)
    m_ref[...] = s / cnt


_mred = pl.pallas_call(
    _mred_body,
    out_shape=jax.ShapeDtypeStruct((_RT, 16), jnp.float32),
)


def _sc2_body(idx_hbm, t_hbm, ev_hbm, er_hbm, m_hbm,
              evc_hbm, erc_hbm, t3_hbm,
              idxw, tw, means, evb, erb, evcb, ercb, t3b):
    cid = lax.axis_index("c")
    sid = lax.axis_index("s")
    wid, base_sub, nsub = _tile_range(cid, sid)
    pltpu.sync_copy(m_hbm, means)

    def center_sub(lb):
        idx16 = idxw[pl.ds(lb, 16)]
        t16 = tw[pl.ds(lb, 16)]
        ti = idx16 * 16
        for c in range(3):
            mv = plsc.load_gather(means, [ti + c])
            mr = plsc.load_gather(means, [ti + (3 + c)])
            evc = evb[pl.ds(c * 1024 + lb, 16)]
            erc = erb[pl.ds(c * 1024 + lb, 16)]
            evcb[pl.ds(c * 1024 + lb, 16)] = evc - mv
            ercb[pl.ds(c * 1024 + lb, 16)] = erc - mr
            t3b[pl.ds(c * 1024 + lb, 16)] = t16

    def group_body(g, carry):
        base = (base_sub + g * 64) * 16
        pltpu.sync_copy(idx_hbm.at[pl.ds(base, 1024)], idxw)
        pltpu.sync_copy(t_hbm.at[pl.ds(base, 1024)], tw)
        for c in range(3):
            pltpu.sync_copy(ev_hbm.at[pl.ds(c * _N + base, 1024)],
                            evb.at[pl.ds(c * 1024, 1024)])
            pltpu.sync_copy(er_hbm.at[pl.ds(c * _N + base, 1024)],
                            erb.at[pl.ds(c * 1024, 1024)])
        for s in range(64):
            center_sub(s * 16)
        for c in range(3):
            pltpu.sync_copy(evcb.at[pl.ds(c * 1024, 1024)],
                            evc_hbm.at[pl.ds(c * _N + base, 1024)])
            pltpu.sync_copy(ercb.at[pl.ds(c * 1024, 1024)],
                            erc_hbm.at[pl.ds(c * _N + base, 1024)])
            pltpu.sync_copy(t3b.at[pl.ds(c * 1024, 1024)],
                            t3_hbm.at[pl.ds(c * _N + base, 1024)])
        return carry

    lax.fori_loop(0, _NGF, group_body, 0)

    def tail_body(s2, carry):
        base = (base_sub + s2) * 16
        pltpu.sync_copy(idx_hbm.at[pl.ds(base, 16)], idxw.at[pl.ds(0, 16)])
        pltpu.sync_copy(t_hbm.at[pl.ds(base, 16)], tw.at[pl.ds(0, 16)])
        for c in range(3):
            pltpu.sync_copy(ev_hbm.at[pl.ds(c * _N + base, 16)],
                            evb.at[pl.ds(c * 1024, 16)])
            pltpu.sync_copy(er_hbm.at[pl.ds(c * _N + base, 16)],
                            erb.at[pl.ds(c * 1024, 16)])
        center_sub(0)
        for c in range(3):
            pltpu.sync_copy(evcb.at[pl.ds(c * 1024, 16)],
                            evc_hbm.at[pl.ds(c * _N + base, 16)])
            pltpu.sync_copy(ercb.at[pl.ds(c * 1024, 16)],
                            erc_hbm.at[pl.ds(c * _N + base, 16)])
            pltpu.sync_copy(t3b.at[pl.ds(c * 1024, 16)],
                            t3_hbm.at[pl.ds(c * _N + base, 16)])
        return carry

    lax.fori_loop(_NGF * 64, nsub, tail_body, 0)


_sc2 = pl.kernel(
    _sc2_body,
    out_type=(
        jax.ShapeDtypeStruct((3 * _N,), jnp.float32),
        jax.ShapeDtypeStruct((3 * _N,), jnp.float32),
        jax.ShapeDtypeStruct((3 * _N,), jnp.float32),
    ),
    mesh=_sc_mesh,
    compiler_params=pltpu.CompilerParams(needs_layout_passes=False),
    scratch_types=[
        pltpu.VMEM((1024,), jnp.int32),
        pltpu.VMEM((1024,), jnp.float32),
        pltpu.VMEM((_RT * 16,), jnp.float32),
        pltpu.VMEM((3072,), jnp.float32),
        pltpu.VMEM((3072,), jnp.float32),
        pltpu.VMEM((3072,), jnp.float32),
        pltpu.VMEM((3072,), jnp.float32),
        pltpu.VMEM((3072,), jnp.float32),
    ],
    name="sc_center_expand",
)


def _tc_body(t3_ref, f0_ref, v0_ref, evc_ref, erc_ref, ft_ref, vt_ref, rt_ref):
    t = _TSCALE * t3_ref[...]
    f0 = f0_ref[...]
    v0 = v0_ref[...]
    ev = evc_ref[...]
    er = erc_ref[...]
    e = jnp.exp(-t)
    sigma_v = jnp.sqrt(jnp.maximum(1.0 - jnp.exp(-2.0 * t), _EPS))
    vt = e * v0 + sigma_v * ev
    coeff = (1.0 - e) / (1.0 + e)
    mu = coeff * (vt + v0)
    sigma_r = jnp.sqrt(jnp.maximum(2.0 * t + 8.0 / (1.0 + jnp.exp(t)) - 4.0,
                                   _EPS))
    rt = _wrap(mu + sigma_r * er)
    ft = _wrap(_wrap(f0) + rt)
    ft_ref[...] = ft
    vt_ref[...] = vt
    rt_ref[...] = rt


_tc = pl.pallas_call(
    _tc_body,
    out_shape=tuple(jax.ShapeDtypeStruct((3 * _N,), jnp.float32)
                    for _ in range(3)),
    grid=(_TCGRID,),
    in_specs=[pl.BlockSpec((_TCBLK,), lambda i: (i,)) for _ in range(5)],
    out_specs=[pl.BlockSpec((_TCBLK,), lambda i: (i,)) for _ in range(3)],
)


def kernel(t, f0, index, v0, epsilon_v, epsilon_r):
    idx = index.astype(jnp.int32)
    evf = epsilon_v.T.reshape(-1)
    erf = epsilon_r.T.reshape(-1)
    f0f = f0.T.reshape(-1)
    v0f = v0.T.reshape(-1)

    partials = _sc1(idx, evf, erf)
    means = _mred(partials.reshape(32, _RT, 16))
    evcf, ercf, t3f = _sc2(idx, t, evf, erf, means.reshape(-1))
    ftf, vtf, rtf = _tc(t3f, f0f, v0f, evcf, ercf)

    def unflat(x):
        return x.reshape(3, _N).T

    return (unflat(ftf), unflat(vtf), unflat(evcf), unflat(ercf), unflat(rtf))


# speedup vs baseline: 18.7122x; 1.2390x over previous
import numpy as np

import jax
import jax.numpy as jnp
from jax import lax
from jax.experimental import pallas as pl
from jax.experimental.pallas import tpu as pltpu
from jax.experimental.pallas import tpu_sc as plsc

_N = 500000
_B = 1024
_EPS = 1e-05
_TSCALE = 2.0

_W0 = 15632
_S0 = _W0 // 16
_S1 = (_N - 31 * _W0) // 16
_NGF = 15
_RT = 1152
_TCBLK = 262144
_TCGRID = (3 * _N + _TCBLK - 1) // _TCBLK


def _wrap(x):
    return jnp.remainder(x + 0.5, 1.0) - 0.5


def _tile_range(cid, sid):
    wid = cid * 16 + sid
    base_sub = wid * _S0
    nsub = jnp.where(wid == 31, _S1, _S0)
    return wid, base_sub, nsub


def _sc1_body(idx_hbm, ev_hbm, er_hbm, out_hbm,
              idxwA, idxwB, evbA, evbB, erbA, erbB, acc, semA, semB):
    cid = lax.axis_index("c")
    sid = lax.axis_index("s")
    wid, base_sub, nsub = _tile_range(cid, sid)
    zeros16 = jnp.zeros((16,), jnp.float32)
    ones16 = jnp.ones((16,), jnp.float32)

    def zero_body(r, carry):
        acc[pl.ds(r * 16, 16)] = zeros16
        return carry

    lax.fori_loop(0, _RT, zero_body, 0)

    def issue(idxw, evb, erb, sem, g):
        base = (base_sub + g * 64) * 16
        pltpu.async_copy(idx_hbm.at[pl.ds(base, 1024)], idxw, sem)
        for c in range(3):
            pltpu.async_copy(ev_hbm.at[pl.ds(c * _N + base, 1024)],
                             evb.at[pl.ds(c * 1024, 1024)], sem)
            pltpu.async_copy(er_hbm.at[pl.ds(c * _N + base, 1024)],
                             erb.at[pl.ds(c * 1024, 1024)], sem)

    def drain(idxw, evb, erb, sem, g):
        base = (base_sub + g * 64) * 16
        pltpu.make_async_copy(idx_hbm.at[pl.ds(base, 1024)], idxw, sem).wait()
        for c in range(3):
            pltpu.make_async_copy(ev_hbm.at[pl.ds(c * _N + base, 1024)],
                                  evb.at[pl.ds(c * 1024, 1024)], sem).wait()
            pltpu.make_async_copy(er_hbm.at[pl.ds(c * _N + base, 1024)],
                                  erb.at[pl.ds(c * 1024, 1024)], sem).wait()

    def sum_sub(idxw, evb, erb, lb):
        idx16 = idxw[pl.ds(lb, 16)]
        ti = idx16 * 16
        for c in range(3):
            evc = evb[pl.ds(c * 1024 + lb, 16)]
            erc = erb[pl.ds(c * 1024 + lb, 16)]
            plsc.addupdate_scatter(acc, [ti + c], evc)
            plsc.addupdate_scatter(acc, [ti + (3 + c)], erc)
        plsc.addupdate_scatter(acc, [ti + 6], ones16)

    def compute(idxw, evb, erb):
        for s in range(64):
            sum_sub(idxw, evb, erb, s * 16)

    issue(idxwA, evbA, erbA, semA, 0)

    def pair_body(h, carry):
        g = h * 2
        issue(idxwB, evbB, erbB, semB, g + 1)
        drain(idxwA, evbA, erbA, semA, g)
        compute(idxwA, evbA, erbA)
        issue(idxwA, evbA, erbA, semA, g + 2)
        drain(idxwB, evbB, erbB, semB, g + 1)
        compute(idxwB, evbB, erbB)
        return carry

    lax.fori_loop(0, 7, pair_body, 0)
    drain(idxwA, evbA, erbA, semA, 14)
    compute(idxwA, evbA, erbA)

    def tail_body(s2, carry):
        base = (base_sub + s2) * 16
        pltpu.sync_copy(idx_hbm.at[pl.ds(base, 16)], idxwA.at[pl.ds(0, 16)])
        for c in range(3):
            pltpu.sync_copy(ev_hbm.at[pl.ds(c * _N + base, 16)],
                            evbA.at[pl.ds(c * 1024, 16)])
            pltpu.sync_copy(er_hbm.at[pl.ds(c * _N + base, 16)],
                            erbA.at[pl.ds(c * 1024, 16)])
        sum_sub(idxwA, evbA, erbA, 0)
        return carry

    lax.fori_loop(_NGF * 64, nsub, tail_body, 0)
    pltpu.sync_copy(acc, out_hbm.at[wid])


_sc_mesh = plsc.VectorSubcoreMesh(core_axis_name="c", subcore_axis_name="s")

_sc1 = pl.kernel(
    _sc1_body,
    out_type=jax.ShapeDtypeStruct((32, _RT * 16), jnp.float32),
    mesh=_sc_mesh,
    compiler_params=pltpu.CompilerParams(needs_layout_passes=False),
    scratch_types=[
        pltpu.VMEM((1024,), jnp.int32),
        pltpu.VMEM((1024,), jnp.int32),
        pltpu.VMEM((3072,), jnp.float32),
        pltpu.VMEM((3072,), jnp.float32),
        pltpu.VMEM((3072,), jnp.float32),
        pltpu.VMEM((3072,), jnp.float32),
        pltpu.VMEM((_RT * 16,), jnp.float32),
        pltpu.SemaphoreType.DMA,
        pltpu.SemaphoreType.DMA,
    ],
    name="sc_segment_sums",
)


def _mred_body(p_ref, m_ref):
    s = jnp.sum(p_ref[...], axis=0)
    cnt = jnp.maximum(s[:, 6:7], 1.0)
    m_ref[...] = s / cnt


_mred = pl.pallas_call(
    _mred_body,
    out_shape=jax.ShapeDtypeStruct((_RT, 16), jnp.float32),
)


def _sc2_body(idx_hbm, t_hbm, ev_hbm, er_hbm, m_hbm,
              evc_hbm, erc_hbm, t3_hbm,
              idxwA, idxwB, twA, twB, evbA, evbB, erbA, erbB,
              means, evcb, ercb, t3b, semA, semB):
    cid = lax.axis_index("c")
    sid = lax.axis_index("s")
    wid, base_sub, nsub = _tile_range(cid, sid)
    pltpu.sync_copy(m_hbm, means)

    def issue(idxw, tw, evb, erb, sem, g):
        base = (base_sub + g * 64) * 16
        pltpu.async_copy(idx_hbm.at[pl.ds(base, 1024)], idxw, sem)
        pltpu.async_copy(t_hbm.at[pl.ds(base, 1024)], tw, sem)
        for c in range(3):
            pltpu.async_copy(ev_hbm.at[pl.ds(c * _N + base, 1024)],
                             evb.at[pl.ds(c * 1024, 1024)], sem)
            pltpu.async_copy(er_hbm.at[pl.ds(c * _N + base, 1024)],
                             erb.at[pl.ds(c * 1024, 1024)], sem)

    def drain(idxw, tw, evb, erb, sem, g):
        base = (base_sub + g * 64) * 16
        pltpu.make_async_copy(idx_hbm.at[pl.ds(base, 1024)], idxw, sem).wait()
        pltpu.make_async_copy(t_hbm.at[pl.ds(base, 1024)], tw, sem).wait()
        for c in range(3):
            pltpu.make_async_copy(ev_hbm.at[pl.ds(c * _N + base, 1024)],
                                  evb.at[pl.ds(c * 1024, 1024)], sem).wait()
            pltpu.make_async_copy(er_hbm.at[pl.ds(c * _N + base, 1024)],
                                  erb.at[pl.ds(c * 1024, 1024)], sem).wait()

    def center_sub(idxw, tw, evb, erb, lb):
        idx16 = idxw[pl.ds(lb, 16)]
        t16 = tw[pl.ds(lb, 16)]
        ti = idx16 * 16
        for c in range(3):
            mv = plsc.load_gather(means, [ti + c])
            mr = plsc.load_gather(means, [ti + (3 + c)])
            evc = evb[pl.ds(c * 1024 + lb, 16)]
            erc = erb[pl.ds(c * 1024 + lb, 16)]
            evcb[pl.ds(c * 1024 + lb, 16)] = evc - mv
            ercb[pl.ds(c * 1024 + lb, 16)] = erc - mr
            t3b[pl.ds(c * 1024 + lb, 16)] = t16

    def compute_out(idxw, tw, evb, erb, g):
        base = (base_sub + g * 64) * 16
        for s in range(64):
            center_sub(idxw, tw, evb, erb, s * 16)
        for c in range(3):
            pltpu.sync_copy(evcb.at[pl.ds(c * 1024, 1024)],
                            evc_hbm.at[pl.ds(c * _N + base, 1024)])
            pltpu.sync_copy(ercb.at[pl.ds(c * 1024, 1024)],
                            erc_hbm.at[pl.ds(c * _N + base, 1024)])
            pltpu.sync_copy(t3b.at[pl.ds(c * 1024, 1024)],
                            t3_hbm.at[pl.ds(c * _N + base, 1024)])

    issue(idxwA, twA, evbA, erbA, semA, 0)

    def pair_body(h, carry):
        g = h * 2
        issue(idxwB, twB, evbB, erbB, semB, g + 1)
        drain(idxwA, twA, evbA, erbA, semA, g)
        compute_out(idxwA, twA, evbA, erbA, g)
        issue(idxwA, twA, evbA, erbA, semA, g + 2)
        drain(idxwB, twB, evbB, erbB, semB, g + 1)
        compute_out(idxwB, twB, evbB, erbB, g + 1)
        return carry

    lax.fori_loop(0, 7, pair_body, 0)
    drain(idxwA, twA, evbA, erbA, semA, 14)
    compute_out(idxwA, twA, evbA, erbA, 14)

    def tail_body(s2, carry):
        base = (base_sub + s2) * 16
        pltpu.sync_copy(idx_hbm.at[pl.ds(base, 16)], idxwA.at[pl.ds(0, 16)])
        pltpu.sync_copy(t_hbm.at[pl.ds(base, 16)], twA.at[pl.ds(0, 16)])
        for c in range(3):
            pltpu.sync_copy(ev_hbm.at[pl.ds(c * _N + base, 16)],
                            evbA.at[pl.ds(c * 1024, 16)])
            pltpu.sync_copy(er_hbm.at[pl.ds(c * _N + base, 16)],
                            erbA.at[pl.ds(c * 1024, 16)])
        center_sub(idxwA, twA, evbA, erbA, 0)
        for c in range(3):
            pltpu.sync_copy(evcb.at[pl.ds(c * 1024, 16)],
                            evc_hbm.at[pl.ds(c * _N + base, 16)])
            pltpu.sync_copy(ercb.at[pl.ds(c * 1024, 16)],
                            erc_hbm.at[pl.ds(c * _N + base, 16)])
            pltpu.sync_copy(t3b.at[pl.ds(c * 1024, 16)],
                            t3_hbm.at[pl.ds(c * _N + base, 16)])
        return carry

    lax.fori_loop(_NGF * 64, nsub, tail_body, 0)


_sc2 = pl.kernel(
    _sc2_body,
    out_type=(
        jax.ShapeDtypeStruct((3 * _N,), jnp.float32),
        jax.ShapeDtypeStruct((3 * _N,), jnp.float32),
        jax.ShapeDtypeStruct((3 * _N,), jnp.float32),
    ),
    mesh=_sc_mesh,
    compiler_params=pltpu.CompilerParams(needs_layout_passes=False),
    scratch_types=[
        pltpu.VMEM((1024,), jnp.int32),
        pltpu.VMEM((1024,), jnp.int32),
        pltpu.VMEM((1024,), jnp.float32),
        pltpu.VMEM((1024,), jnp.float32),
        pltpu.VMEM((3072,), jnp.float32),
        pltpu.VMEM((3072,), jnp.float32),
        pltpu.VMEM((3072,), jnp.float32),
        pltpu.VMEM((3072,), jnp.float32),
        pltpu.VMEM((_RT * 16,), jnp.float32),
        pltpu.VMEM((3072,), jnp.float32),
        pltpu.VMEM((3072,), jnp.float32),
        pltpu.VMEM((3072,), jnp.float32),
        pltpu.SemaphoreType.DMA,
        pltpu.SemaphoreType.DMA,
    ],
    name="sc_center_expand",
)


def _tc_body(t3_ref, f0_ref, v0_ref, evc_ref, erc_ref, ft_ref, vt_ref, rt_ref):
    t = _TSCALE * t3_ref[...]
    f0 = f0_ref[...]
    v0 = v0_ref[...]
    ev = evc_ref[...]
    er = erc_ref[...]
    e = jnp.exp(-t)
    sigma_v = jnp.sqrt(jnp.maximum(1.0 - jnp.exp(-2.0 * t), _EPS))
    vt = e * v0 + sigma_v * ev
    coeff = (1.0 - e) / (1.0 + e)
    mu = coeff * (vt + v0)
    sigma_r = jnp.sqrt(jnp.maximum(2.0 * t + 8.0 / (1.0 + jnp.exp(t)) - 4.0,
                                   _EPS))
    rt = _wrap(mu + sigma_r * er)
    ft = _wrap(_wrap(f0) + rt)
    ft_ref[...] = ft
    vt_ref[...] = vt
    rt_ref[...] = rt


_tc = pl.pallas_call(
    _tc_body,
    out_shape=tuple(jax.ShapeDtypeStruct((3 * _N,), jnp.float32)
                    for _ in range(3)),
    grid=(_TCGRID,),
    in_specs=[pl.BlockSpec((_TCBLK,), lambda i: (i,)) for _ in range(5)],
    out_specs=[pl.BlockSpec((_TCBLK,), lambda i: (i,)) for _ in range(3)],
)


def kernel(t, f0, index, v0, epsilon_v, epsilon_r):
    idx = index.astype(jnp.int32)
    evf = epsilon_v.T.reshape(-1)
    erf = epsilon_r.T.reshape(-1)
    f0f = f0.T.reshape(-1)
    v0f = v0.T.reshape(-1)

    partials = _sc1(idx, evf, erf)
    means = _mred(partials.reshape(32, _RT, 16))
    evcf, ercf, t3f = _sc2(idx, t, evf, erf, means.reshape(-1))
    ftf, vtf, rtf = _tc(t3f, f0f, v0f, evcf, ercf)

    def unflat(x):
        return x.reshape(3, _N).T

    return (unflat(ftf), unflat(vtf), unflat(evcf), unflat(ercf), unflat(rtf))
